# baseline (speedup 1.0000x reference)
import functools

import jax
import jax.numpy as jnp
from jax import lax
from jax.experimental import pallas as pl
from jax.experimental.pallas import tpu as pltpu
from jax.experimental.pallas import tpu_sc as plsc

_NC = 2
_NS = 16
_NW = _NC * _NS
_IDX_CHUNK = 128


def _sc_gather(uid2, gid2, user_table, geo_table):
    n_rows = uid2.shape[0]
    b = n_rows * _IDX_CHUNK
    rows_per_w = n_rows // _NW
    bpw = rows_per_w * _IDX_CHUNK
    du = user_table.shape[1]
    dg = geo_table.shape[1]

    @functools.partial(
        pl.kernel,
        mesh=plsc.VectorSubcoreMesh(core_axis_name="c", subcore_axis_name="s"),
        compiler_params=pltpu.CompilerParams(use_tc_tiling_on_sc=False),
        out_type=[
            jax.ShapeDtypeStruct((b, du), jnp.float32),
            jax.ShapeDtypeStruct((b, dg), jnp.float32),
        ],
        scratch_types=[
            pltpu.VMEM((rows_per_w, _IDX_CHUNK), jnp.int32),
            pltpu.VMEM((rows_per_w, _IDX_CHUNK), jnp.int32),
            pltpu.VMEM((bpw, du), jnp.float32),
            pltpu.VMEM((bpw, dg), jnp.float32),
            pltpu.SemaphoreType.DMA,
        ],
    )
    def gather_kernel(uid_hbm, gid_hbm, utab_hbm, gtab_hbm, uout_hbm, gout_hbm,
                      uidx, gidx, urows, grows, sem):
        wid = lax.axis_index("s") * _NC + lax.axis_index("c")
        pltpu.sync_copy(uid_hbm.at[pl.ds(wid * rows_per_w, rows_per_w)], uidx)
        pltpu.sync_copy(gid_hbm.at[pl.ds(wid * rows_per_w, rows_per_w)], gidx)
        copies = []
        for j in range(rows_per_w):
            copies.append(pltpu.async_copy(
                utab_hbm.at[uidx.at[j]],
                urows.at[pl.ds(j * _IDX_CHUNK, _IDX_CHUNK)], sem))
            copies.append(pltpu.async_copy(
                gtab_hbm.at[gidx.at[j]],
                grows.at[pl.ds(j * _IDX_CHUNK, _IDX_CHUNK)], sem))
        for c in copies:
            c.wait()
        pltpu.sync_copy(urows, uout_hbm.at[pl.ds(wid * bpw, bpw)])
        pltpu.sync_copy(grows, gout_hbm.at[pl.ds(wid * bpw, bpw)])

    return gather_kernel(uid2, gid2, user_table, geo_table)


def _mlp_body(ue_ref, ge_ref, ab_ref, sb_ref, iv_ref, at_ref, st_ref,
              w0u_ref, w0g_ref, w0a_ref, w0s_ref, w0i_ref, b0_ref,
              w1_ref, b1_ref, w2_ref, b2_ref, out_ref):
    f32 = jnp.float32
    tile = ue_ref.shape[0]
    dot = functools.partial(jnp.dot, preferred_element_type=f32)

    a_onehot = (ab_ref[...] == lax.broadcasted_iota(jnp.int32, (tile, 16), 1)
                ).astype(f32)
    s_onehot = (sb_ref[...] == lax.broadcasted_iota(jnp.int32, (tile, 16), 1)
                ).astype(f32)
    a_fold = dot(at_ref[...], w0a_ref[...])
    s_fold = dot(st_ref[...], w0s_ref[...])

    h = (dot(ue_ref[...], w0u_ref[...])
         + dot(ge_ref[...], w0g_ref[...])
         + dot(iv_ref[...], w0i_ref[...])
         + dot(a_onehot, a_fold)
         + dot(s_onehot, s_fold)
         + b0_ref[...])
    h = jnp.maximum(h, 0.0)
    h = jnp.maximum(dot(h, w1_ref[...]) + b1_ref[...], 0.0)
    o = dot(h, w2_ref[...]) + b2_ref[...]
    n2 = jnp.sum(o * o, axis=1, keepdims=True)
    out_ref[...] = o * lax.rsqrt(jnp.maximum(n2, 1e-24))


def _tc_mlp(user_emb, geo_emb, age_b, sched_b, interest,
            age_pad, sched_pad, W0u, W0g, W0a, W0s, W0i, b0, W1, b1, W2, b2):
    b = user_emb.shape[0]
    tile = 2048
    grid = (b // tile,)
    d_out = W2.shape[1]

    def rowblk(cols):
        return pl.BlockSpec((tile, cols), lambda i: (i, 0))

    def full(shape):
        return pl.BlockSpec(shape, lambda i: (0, 0))

    return pl.pallas_call(
        _mlp_body,
        grid=grid,
        in_specs=[
            rowblk(user_emb.shape[1]),
            rowblk(geo_emb.shape[1]),
            rowblk(1),
            rowblk(1),
            rowblk(interest.shape[1]),
            full(age_pad.shape),
            full(sched_pad.shape),
            full(W0u.shape),
            full(W0g.shape),
            full(W0a.shape),
            full(W0s.shape),
            full(W0i.shape),
            full(b0.shape),
            full(W1.shape),
            full(b1.shape),
            full(W2.shape),
            full(b2.shape),
        ],
        out_specs=rowblk(d_out),
        out_shape=jax.ShapeDtypeStruct((b, d_out), jnp.float32),
    )(user_emb, geo_emb, age_b, sched_b, interest,
      age_pad, sched_pad, W0u, W0g, W0a, W0s, W0i, b0, W1, b1, W2, b2)


def kernel(user_ids, geo_cells, age_buckets, schedule_types, interest_vectors,
           user_table, geo_table, age_table, sched_table,
           W0, b0, W1, b1, W2, b2):
    uid2 = user_ids.astype(jnp.int32).reshape(-1, _IDX_CHUNK)
    gid2 = geo_cells.astype(jnp.int32).reshape(-1, _IDX_CHUNK)
    user_emb, geo_emb = _sc_gather(uid2, gid2, user_table, geo_table)

    du = user_table.shape[1]
    dg = geo_table.shape[1]
    da = age_table.shape[1]
    ds_ = sched_table.shape[1]
    di = interest_vectors.shape[1]
    o1 = du
    o2 = o1 + dg
    o3 = o2 + da
    o4 = o3 + ds_
    W0u = W0[:o1]
    W0g = W0[o1:o2]
    W0a = W0[o2:o3]
    W0s = W0[o3:o4]
    W0i = W0[o4:o4 + di]

    age_pad = jnp.zeros((16, da), jnp.float32).at[:age_table.shape[0]].set(age_table)
    sched_pad = jnp.zeros((16, ds_), jnp.float32).at[:sched_table.shape[0]].set(sched_table)

    return _tc_mlp(
        user_emb, geo_emb,
        age_buckets.astype(jnp.int32).reshape(-1, 1),
        schedule_types.astype(jnp.int32).reshape(-1, 1),
        interest_vectors,
        age_pad, sched_pad,
        W0u, W0g, W0a, W0s, W0i,
        b0.reshape(1, -1), W1, b1.reshape(1, -1), W2, b2.reshape(1, -1))
